# Initial kernel scaffold; baseline (speedup 1.0000x reference)
import jax
import jax.numpy as jnp
from jax.experimental import pallas as pl


def kernel(data_points, clustering):
    raise NotImplementedError("write your pallas kernel here")


# speedup vs baseline: 6.4699x; 6.4699x over previous
import functools

import jax
import jax.numpy as jnp
from jax import lax
from jax.experimental import pallas as pl
from jax.experimental.pallas import tpu as pltpu
from jax.experimental.pallas import tpu_sc as plsc

K = 64
DIM = 64
NC = 2
NS = 16
L = 16
NW = NC * NS
G = 128
SUPER = 1024


def _sc_moments_body(x_hbm, ids_hbm, out_s, out_q, rows, qrows, ids, acc_s,
                     acc_q):
    cid = lax.axis_index("c")
    sid = lax.axis_index("s")
    wid = cid * NS + sid
    n = x_hbm.shape[0]
    chunk = n // NW
    n_super = chunk // SUPER

    zero = jnp.zeros((L,), jnp.float32)
    lane0_one = jnp.where(lax.iota(jnp.int32, L) == 0,
                          jnp.float32(1.0), jnp.float32(0.0))

    def _zrow(r, _):
        for j in range(DIM // L):
            rows[r, pl.ds(j * L, L)] = zero
        qrows[r, pl.ds(0, L)] = zero
        qrows[r, pl.ds(L, L)] = zero
        return 0

    lax.fori_loop(0, K, _zrow, 0)

    @pl.when(sid == 0)
    def _():
        pltpu.sync_copy(rows.at[pl.ds(0, K)], acc_s)
        pltpu.sync_copy(qrows.at[pl.ds(0, K)], acc_q)

    def _prow(p, _):
        qrows[p, pl.ds(L, L)] = lane0_one
        return 0

    lax.fori_loop(0, SUPER, _prow, 0)
    plsc.subcore_barrier()

    def _super(si, _):
        base = pl.multiple_of(wid * chunk + si * SUPER, SUPER)
        pltpu.sync_copy(x_hbm.at[pl.ds(base, SUPER)], rows)
        pltpu.sync_copy(ids_hbm.at[pl.ds(pl.multiple_of(base // G, 8),
                                         SUPER // G)], ids)

        @plsc.parallel_loop(0, SUPER, 1, unroll=8)
        def _q(p):
            v = rows[p, pl.ds(0, L)]
            acc = v * v
            for j in range(1, DIM // L):
                v = rows[p, pl.ds(j * L, L)]
                acc = acc + v * v
            qrows[p, pl.ds(0, L)] = acc

        for g in range(SUPER // G):
            idx = ids.at[g]
            pltpu.sync_copy(rows.at[pl.ds(g * G, G)], acc_s.at[idx], add=True)
            pltpu.sync_copy(qrows.at[pl.ds(g * G, G)], acc_q.at[idx], add=True)
        return 0

    lax.fori_loop(0, n_super, _super, 0)
    plsc.subcore_barrier()

    @pl.when(sid == 0)
    def _():
        pltpu.sync_copy(acc_s, out_s.at[cid])
        pltpu.sync_copy(acc_q, out_q.at[cid])


def _sc_moments(data_points, ids2d):
    mesh = plsc.VectorSubcoreMesh(core_axis_name="c", subcore_axis_name="s",
                                  num_cores=NC, num_subcores=NS)
    f = pl.kernel(
        _sc_moments_body,
        out_type=[
            jax.ShapeDtypeStruct((NC, K, DIM), jnp.float32),
            jax.ShapeDtypeStruct((NC, K, 2 * L), jnp.float32),
        ],
        mesh=mesh,
        scratch_types=[
            pltpu.VMEM((SUPER, DIM), jnp.float32),
            pltpu.VMEM((SUPER, 2 * L), jnp.float32),
            pltpu.VMEM((SUPER // G, G), jnp.int32),
            pltpu.VMEM_SHARED((K, DIM), jnp.float32),
            pltpu.VMEM_SHARED((K, 2 * L), jnp.float32),
        ],
        compiler_params=pltpu.CompilerParams(use_tc_tiling_on_sc=False),
    )
    return f(data_points, ids2d)


def _finalize_body(s_ref, q_ref, o_ref):
    s = s_ref[0] + s_ref[1]
    qr = q_ref[0] + q_ref[1]
    q = jnp.sum(qr[:, :L], axis=1, keepdims=True)
    m = qr[:, L:L + 1]
    cnt = m + 1.0
    ai = (s + 0.001) / cnt
    si_sum = (0.001 + q
              - 2.0 * jnp.sum(ai * s, axis=1, keepdims=True)
              + m * jnp.sum(ai * ai, axis=1, keepdims=True))
    si = jnp.sqrt(si_sum / cnt)

    diff = ai[:, None, :] - ai[None, :, :]
    mij = jnp.sqrt(jnp.sum(diff * diff, axis=-1))

    ones = jnp.ones((K, 1), jnp.float32)
    si_j = lax.dot_general(ones, si, (((1,), (1,)), ((), ())),
                           preferred_element_type=jnp.float32)
    rsum = si + si_j
    safe_m = jnp.where(mij == 0.0, 1.0, mij)
    rij = jnp.where(mij == 0.0, 0.1, rsum / safe_m)
    ii = lax.broadcasted_iota(jnp.int32, (K, K), 0)
    jj = lax.broadcasted_iota(jnp.int32, (K, K), 1)
    rij = jnp.where(ii == jj, 0.0, rij)
    di = jnp.max(rij, axis=1, keepdims=True)
    o_ref[...] = jnp.sum(di, axis=0, keepdims=True) / jnp.float32(K)


def _finalize(part_s, part_q):
    return pl.pallas_call(
        _finalize_body,
        out_shape=jax.ShapeDtypeStruct((1, 1), jnp.float32),
    )(part_s, part_q)


@jax.jit
def kernel(data_points, clustering):
    n = data_points.shape[0]
    ids2d = clustering.reshape(n // G, G)
    part_s, part_q = _sc_moments(data_points, ids2d)
    out = _finalize(part_s, part_q)
    return out[0, 0]


# speedup vs baseline: 11.7757x; 1.8201x over previous
import jax
import jax.numpy as jnp
from jax import lax
from jax.experimental import pallas as pl
from jax.experimental.pallas import tpu as pltpu
from jax.experimental.pallas import tpu_sc as plsc

K = 64
DIM = 64
NC = 2
NS = 16
L = 16
NW = NC * NS
N = 131072
CHUNK = N // NW
WIN = 2048
CS = DIM + 2
ROW = L * CS


def _sc_body(xt_hbm, ids_hbm, out, band, ids, acc):
    cid = lax.axis_index("c")
    sid = lax.axis_index("s")
    wid = cid * NS + sid

    zero = jnp.zeros((L,), jnp.float32)
    ones = jnp.full((L,), 1.0, jnp.float32)
    laneoff = lax.iota(jnp.int32, L) * jnp.int32(CS)

    def _zrow(r, _):
        for j in range(ROW // L):
            acc[r, pl.ds(j * L, L)] = zero
        return 0

    lax.fori_loop(0, K, _zrow, 0)

    pltpu.sync_copy(
        ids_hbm.at[pl.ds(pl.multiple_of(wid * CHUNK, CHUNK), CHUNK)], ids)

    for w in range(CHUNK // WIN):
        nbase = pl.multiple_of(wid * CHUNK + w * WIN, WIN)
        for b in range(DIM // 8):
            pltpu.sync_copy(xt_hbm.at[pl.ds(b * 8, 8), pl.ds(nbase, WIN)],
                            band)

            @plsc.parallel_loop(0, WIN // L, 1, unroll=2)
            def _g(g):
                cvec = ids[pl.ds(w * WIN + g * L, L)]
                col = laneoff
                qv = zero
                v = band[0, pl.ds(g * L, L)]
                plsc.addupdate_scatter(acc, [cvec, col + jnp.int32(b * 8)], v)
                qv = v * v
                for r in range(1, 8):
                    v = band[r, pl.ds(g * L, L)]
                    plsc.addupdate_scatter(
                        acc, [cvec, col + jnp.int32(b * 8 + r)], v)
                    qv = qv + v * v
                plsc.addupdate_scatter(acc, [cvec, col + jnp.int32(DIM)], qv)
                if b == 0:
                    plsc.addupdate_scatter(
                        acc, [cvec, col + jnp.int32(DIM + 1)], ones)

    pltpu.sync_copy(acc, out.at[pl.ds(pl.multiple_of(wid * K, K), K)])


def _sc_moments(xt, clustering):
    mesh = plsc.VectorSubcoreMesh(core_axis_name="c", subcore_axis_name="s",
                                  num_cores=NC, num_subcores=NS)
    f = pl.kernel(
        _sc_body,
        out_type=[jax.ShapeDtypeStruct((NW * K, ROW), jnp.float32)],
        mesh=mesh,
        scratch_types=[
            pltpu.VMEM((8, WIN), jnp.float32),
            pltpu.VMEM((CHUNK,), jnp.int32),
            pltpu.VMEM((K, ROW), jnp.float32),
        ],
        compiler_params=pltpu.CompilerParams(use_tc_tiling_on_sc=True,
                                             needs_layout_passes=False),
    )
    return f(xt, clustering)


def _finalize_body(p_ref, o_ref):
    s = jnp.sum(p_ref[...], axis=0)
    a66 = s[:, 0:CS]
    for l in range(1, L):
        a66 = a66 + s[:, l * CS:(l + 1) * CS]
    sx = a66[:, :DIM]
    q = a66[:, DIM:DIM + 1]
    m = a66[:, DIM + 1:DIM + 2]
    cnt = m + 1.0
    ai = (sx + 0.001) / cnt
    si_sum = (0.001 + q
              - 2.0 * jnp.sum(ai * sx, axis=1, keepdims=True)
              + m * jnp.sum(ai * ai, axis=1, keepdims=True))
    si = jnp.sqrt(si_sum / cnt)

    diff = ai[:, None, :] - ai[None, :, :]
    mij = jnp.sqrt(jnp.sum(diff * diff, axis=-1))
    ones = jnp.ones((K, 1), jnp.float32)
    si_j = lax.dot_general(ones, si, (((1,), (1,)), ((), ())),
                           preferred_element_type=jnp.float32)
    rsum = si + si_j
    safe_m = jnp.where(mij == 0.0, 1.0, mij)
    rij = jnp.where(mij == 0.0, 0.1, rsum / safe_m)
    ii = lax.broadcasted_iota(jnp.int32, (K, K), 0)
    jj = lax.broadcasted_iota(jnp.int32, (K, K), 1)
    rij = jnp.where(ii == jj, 0.0, rij)
    di = jnp.max(rij, axis=1, keepdims=True)
    o_ref[...] = jnp.sum(di, axis=0, keepdims=True) / jnp.float32(K)


def _finalize(partials):
    return pl.pallas_call(
        _finalize_body,
        out_shape=jax.ShapeDtypeStruct((1, 1), jnp.float32),
    )(partials)


@jax.jit
def kernel(data_points, clustering):
    (partials,) = _sc_moments(data_points.T, clustering)
    out = _finalize(partials.reshape(NW, K, ROW))
    return out[0, 0]
